# Initial kernel scaffold; baseline (speedup 1.0000x reference)
import jax
import jax.numpy as jnp
from jax.experimental import pallas as pl


def kernel(points, features, coors, f_cluster, params):
    raise NotImplementedError("write your pallas kernel here")


# speedup vs baseline: 1.0393x; 1.0393x over previous
import jax
import jax.numpy as jnp
from jax.experimental import pallas as pl
from jax.experimental.pallas import tpu as pltpu

_EPS = 1e-3
_P = 2560
_NV = 10000
_NEG = float(jnp.finfo(jnp.float32).min)


def _ln(x, g, b):
    m = x.mean(-1, keepdims=True)
    v = ((x - m) ** 2).mean(-1, keepdims=True)
    return (x - m) / jnp.sqrt(v + _EPS) * g + b


def _seg_scan_max(x, seg, reverse):
    n, f = x.shape
    d = 1
    while d < n:
        if not reverse:
            xs = jnp.concatenate([jnp.full((d, f), _NEG, x.dtype), x[:-d]], 0)
            ss = jnp.concatenate([jnp.full((d, 1), -1, seg.dtype), seg[:-d]], 0)
        else:
            xs = jnp.concatenate([x[d:], jnp.full((d, f), _NEG, x.dtype)], 0)
            ss = jnp.concatenate([seg[d:], jnp.full((d, 1), -1, seg.dtype)], 0)
        x = jnp.maximum(x, jnp.where(ss == seg, xs, _NEG))
        d *= 2
    return x


def _stage1_fwd_kernel(has_b, seg_ref, pts_ref, a_ref, b_ref, fcl_ref,
                       w0p_ref, w0a_ref, w0b_ref, b0_ref, g0_ref, bt0_ref,
                       rw1_ref, rb1_ref, rw2_ref, rb2_ref,
                       x_out, fx_out, cvec, cseg):
    @pl.when(pl.program_id(0) == 0)
    def _():
        cvec[...] = jnp.full((1, 32), _NEG, jnp.float32)
        cseg[0] = -1

    seg = seg_ref[...]
    dot = lambda a, w: jnp.dot(a, w, preferred_element_type=jnp.float32)
    pre = dot(pts_ref[...], w0p_ref[...]) + dot(a_ref[...], w0a_ref[...])
    if has_b:
        pre = pre + dot(b_ref[...], w0b_ref[...])
    pre = pre + b0_ref[...]
    x = jnp.maximum(_ln(pre, g0_ref[...], bt0_ref[...]), 0.0)
    rel = jnp.maximum(dot(fcl_ref[...] / 10.0, rw1_ref[...]) + rb1_ref[...], 0.0)
    rel = dot(rel, rw2_ref[...]) + rb2_ref[...]
    x = x + rel
    x_out[...] = x

    fx = _seg_scan_max(x, seg, reverse=False)
    fx = jnp.where(seg == cseg[0], jnp.maximum(fx, cvec[...]), fx)
    fx_out[...] = fx
    cvec[...] = fx[-1:, :]
    cseg[0] = seg[-1, 0]


def _stage2_bwd_kernel(seg_ref, x_ref, fx_ref,
                       w1a_ref, w1b_ref, b1_ref, g1_ref, bt1_ref,
                       y_out, cvec, cseg):
    @pl.when(pl.program_id(0) == 0)
    def _():
        cvec[...] = jnp.full((1, 32), _NEG, jnp.float32)
        cseg[0] = -1

    seg = seg_ref[...]
    x = x_ref[...]
    bx = _seg_scan_max(x, seg, reverse=True)
    bx = jnp.where(seg == cseg[0], jnp.maximum(bx, cvec[...]), bx)
    cvec[...] = bx[:1, :]
    cseg[0] = seg[0, 0]

    c0 = jnp.maximum(fx_ref[...], bx)
    dot = lambda a, w: jnp.dot(a, w, preferred_element_type=jnp.float32)
    pre = dot(x, w1a_ref[...]) + dot(c0, w1b_ref[...]) + b1_ref[...]
    y_out[...] = jnp.maximum(_ln(pre, g1_ref[...], bt1_ref[...]), 0.0)


def _fwd_scan_kernel(seg_ref, y_ref, fy_out, cvec, cseg):
    @pl.when(pl.program_id(0) == 0)
    def _():
        cvec[...] = jnp.full((1, 32), _NEG, jnp.float32)
        cseg[0] = -1
    seg = seg_ref[...]
    fy = _seg_scan_max(y_ref[...], seg, reverse=False)
    fy = jnp.where(seg == cseg[0], jnp.maximum(fy, cvec[...]), fy)
    fy_out[...] = fy
    cvec[...] = fy[-1:, :]
    cseg[0] = seg[-1, 0]


def _bwd_combine_kernel(seg_ref, y_ref, fy_ref, c_out, cvec, cseg):
    @pl.when(pl.program_id(0) == 0)
    def _():
        cvec[...] = jnp.full((1, 32), _NEG, jnp.float32)
        cseg[0] = -1
    seg = seg_ref[...]
    by = _seg_scan_max(y_ref[...], seg, reverse=True)
    by = jnp.where(seg == cseg[0], jnp.maximum(by, cvec[...]), by)
    cvec[...] = by[:1, :]
    cseg[0] = seg[0, 0]
    c_out[...] = jnp.maximum(fy_ref[...], by)


def _full(shape):
    return pl.BlockSpec(shape, lambda c: tuple(0 for _ in shape))


def _chunk(width, rev, nc):
    if rev:
        return pl.BlockSpec((_P, width), lambda c: (nc - 1 - c, 0))
    return pl.BlockSpec((_P, width), lambda c: (c, 0))


_SCRATCH = None


def _scratch():
    return [pltpu.VMEM((1, 32), jnp.float32), pltpu.SMEM((1,), jnp.int32)]


def _params():
    return pltpu.CompilerParams(dimension_semantics=("arbitrary",))


def kernel(points, features, coors, f_cluster, params):
    n = points.shape[0]
    nc = n // _P
    coors = coors.astype(jnp.int32)

    is_new = jnp.concatenate(
        [jnp.ones((1,), jnp.int32), (coors[1:] != coors[:-1]).astype(jnp.int32)])
    unq_inv = (jnp.cumsum(is_new) - 1).astype(jnp.int32)
    n_act = unq_inv[-1] + 1
    unq = jnp.full((_NV,), coors[0], coors.dtype).at[unq_inv].set(coors)
    seg_starts = jnp.searchsorted(coors, unq).astype(jnp.int32)
    seg2d = unq_inv.reshape(n, 1)

    f32 = jnp.float32
    out32 = jax.ShapeDtypeStruct((n, 32), f32)

    def block(p, feats_a, feats_b):
        has_b = feats_b is not None
        wa = feats_a.shape[1]
        w0 = p['W0']
        w0p, w0a = w0[:4], w0[4:4 + wa]
        w0b = w0[4 + wa:] if has_b else jnp.zeros((32, 32), f32)
        fb = feats_b if has_b else jnp.zeros((n, 32), f32)
        row = lambda v: v.reshape(1, -1)

        x, fx = pl.pallas_call(
            lambda *rs: _stage1_fwd_kernel(has_b, *rs),
            grid=(nc,),
            in_specs=[_chunk(1, False, nc), _chunk(4, False, nc),
                      _chunk(wa, False, nc), _chunk(32, False, nc),
                      _chunk(3, False, nc),
                      _full((4, 32)), _full((wa, 32)), _full((32, 32)),
                      _full((1, 32)), _full((1, 32)), _full((1, 32)),
                      _full((3, 16)), _full((1, 16)), _full((16, 32)),
                      _full((1, 32))],
            out_specs=[_chunk(32, False, nc), _chunk(32, False, nc)],
            out_shape=[out32, out32],
            scratch_shapes=_scratch(),
            compiler_params=_params(),
        )(seg2d, points, feats_a, fb, f_cluster,
          w0p, w0a, w0b, row(p['b0']), row(p['g0']), row(p['bt0']),
          p['rel_W1'], row(p['rel_b1']), p['rel_W2'], row(p['rel_b2']))

        y = pl.pallas_call(
            _stage2_bwd_kernel,
            grid=(nc,),
            in_specs=[_chunk(1, True, nc), _chunk(32, True, nc),
                      _chunk(32, True, nc),
                      _full((32, 32)), _full((32, 32)),
                      _full((1, 32)), _full((1, 32)), _full((1, 32))],
            out_specs=[_chunk(32, True, nc)],
            out_shape=[out32],
            scratch_shapes=_scratch(),
            compiler_params=_params(),
        )(seg2d, x, fx, p['W1'][:32], p['W1'][32:],
          row(p['b1']), row(p['g1']), row(p['bt1']))[0]

        fy = pl.pallas_call(
            _fwd_scan_kernel,
            grid=(nc,),
            in_specs=[_chunk(1, False, nc), _chunk(32, False, nc)],
            out_specs=[_chunk(32, False, nc)],
            out_shape=[out32],
            scratch_shapes=_scratch(),
            compiler_params=_params(),
        )(seg2d, y)[0]

        c1 = pl.pallas_call(
            _bwd_combine_kernel,
            grid=(nc,),
            in_specs=[_chunk(1, True, nc), _chunk(32, True, nc),
                      _chunk(32, True, nc)],
            out_specs=[_chunk(32, True, nc)],
            out_shape=[out32],
            scratch_shapes=_scratch(),
            compiler_params=_params(),
        )(seg2d, y, fy)[0]

        return y, c1

    feats_a, feats_b = features, None
    pooled_pts = []
    for p in params:
        y, c1 = block(p, feats_a, feats_b)
        pooled_pts.append(c1)
        feats_a, feats_b = y, c1

    out_feats = jnp.concatenate([feats_a, feats_b], axis=1)
    pooled = jnp.concatenate(
        [jnp.take(c, seg_starts, axis=0) for c in pooled_pts], axis=1)
    mask = (jnp.arange(_NV) < n_act)[:, None]
    final_cluster_feats = jnp.where(mask, pooled, -jnp.inf)
    return (out_feats, final_cluster_feats, unq)


# speedup vs baseline: 2.7140x; 2.6113x over previous
import jax
import jax.numpy as jnp
from jax.experimental import pallas as pl
from jax.experimental.pallas import tpu as pltpu

_EPS = 1e-3
_P = 6400
_NV = 10000
_NEG = float(jnp.finfo(jnp.float32).min)


def _ln_t(x, g, b):
    m = x.mean(0, keepdims=True)
    v = ((x - m) ** 2).mean(0, keepdims=True)
    return (x - m) / jnp.sqrt(v + _EPS) * g + b


def _seg_scan_max_t(x, seg, reverse):
    f, n = x.shape
    d = 1
    while d < n:
        if not reverse:
            xs = jnp.concatenate([jnp.full((f, d), _NEG, x.dtype), x[:, :-d]], 1)
            ss = jnp.concatenate([jnp.full((1, d), -1, seg.dtype), seg[:, :-d]], 1)
        else:
            xs = jnp.concatenate([x[:, d:], jnp.full((f, d), _NEG, x.dtype)], 1)
            ss = jnp.concatenate([seg[:, d:], jnp.full((1, d), -1, seg.dtype)], 1)
        x = jnp.maximum(x, jnp.where(ss == seg, xs, _NEG))
        d *= 2
    return x


def _stage1_fwd_kernel(has_b, seg_ref, pts_ref, a_ref, b_ref, fcl_ref,
                       w0p_ref, w0a_ref, w0b_ref, b0_ref, g0_ref, bt0_ref,
                       rw1_ref, rb1_ref, rw2_ref, rb2_ref,
                       x_out, fx_out, cvec, cseg):
    @pl.when(pl.program_id(0) == 0)
    def _():
        cvec[...] = jnp.full((32, 1), _NEG, jnp.float32)
        cseg[0] = -1

    seg = seg_ref[...]
    dot = lambda w, a: jnp.dot(w, a, preferred_element_type=jnp.float32)
    pre = dot(w0p_ref[...], pts_ref[...]) + dot(w0a_ref[...], a_ref[...])
    if has_b:
        pre = pre + dot(w0b_ref[...], b_ref[...])
    pre = pre + b0_ref[...]
    x = jnp.maximum(_ln_t(pre, g0_ref[...], bt0_ref[...]), 0.0)
    rel = jnp.maximum(dot(rw1_ref[...], fcl_ref[...] / 10.0) + rb1_ref[...], 0.0)
    rel = dot(rw2_ref[...], rel) + rb2_ref[...]
    x = x + rel
    x_out[...] = x

    fx = _seg_scan_max_t(x, seg, reverse=False)
    fx = jnp.where(seg == cseg[0], jnp.maximum(fx, cvec[...]), fx)
    fx_out[...] = fx
    cvec[...] = fx[:, -1:]
    cseg[0] = seg[0, -1]


def _stage2_bwd_kernel(seg_ref, x_ref, fx_ref,
                       w1a_ref, w1b_ref, b1_ref, g1_ref, bt1_ref,
                       y_out, cvec, cseg):
    @pl.when(pl.program_id(0) == 0)
    def _():
        cvec[...] = jnp.full((32, 1), _NEG, jnp.float32)
        cseg[0] = -1

    seg = seg_ref[...]
    x = x_ref[...]
    bx = _seg_scan_max_t(x, seg, reverse=True)
    bx = jnp.where(seg == cseg[0], jnp.maximum(bx, cvec[...]), bx)
    cvec[...] = bx[:, :1]
    cseg[0] = seg[0, 0]

    c0 = jnp.maximum(fx_ref[...], bx)
    dot = lambda w, a: jnp.dot(w, a, preferred_element_type=jnp.float32)
    pre = dot(w1a_ref[...], x) + dot(w1b_ref[...], c0) + b1_ref[...]
    y_out[...] = jnp.maximum(_ln_t(pre, g1_ref[...], bt1_ref[...]), 0.0)


def _fwd_scan_kernel(seg_ref, y_ref, fy_out, cvec, cseg):
    @pl.when(pl.program_id(0) == 0)
    def _():
        cvec[...] = jnp.full((32, 1), _NEG, jnp.float32)
        cseg[0] = -1
    seg = seg_ref[...]
    fy = _seg_scan_max_t(y_ref[...], seg, reverse=False)
    fy = jnp.where(seg == cseg[0], jnp.maximum(fy, cvec[...]), fy)
    fy_out[...] = fy
    cvec[...] = fy[:, -1:]
    cseg[0] = seg[0, -1]


def _bwd_combine_kernel(seg_ref, y_ref, fy_ref, c_out, cvec, cseg):
    @pl.when(pl.program_id(0) == 0)
    def _():
        cvec[...] = jnp.full((32, 1), _NEG, jnp.float32)
        cseg[0] = -1
    seg = seg_ref[...]
    by = _seg_scan_max_t(y_ref[...], seg, reverse=True)
    by = jnp.where(seg == cseg[0], jnp.maximum(by, cvec[...]), by)
    cvec[...] = by[:, :1]
    cseg[0] = seg[0, 0]
    c_out[...] = jnp.maximum(fy_ref[...], by)


def _full(shape):
    return pl.BlockSpec(shape, lambda c: tuple(0 for _ in shape))


def _chunk(height, rev, nc):
    if rev:
        return pl.BlockSpec((height, _P), lambda c: (0, nc - 1 - c))
    return pl.BlockSpec((height, _P), lambda c: (0, c))


def _scratch():
    return [pltpu.VMEM((32, 1), jnp.float32), pltpu.SMEM((1,), jnp.int32)]


def _cparams():
    return pltpu.CompilerParams(dimension_semantics=("arbitrary",))


def kernel(points, features, coors, f_cluster, params):
    n = points.shape[0]
    nc = n // _P
    coors = coors.astype(jnp.int32)

    is_new = jnp.concatenate(
        [jnp.ones((1,), jnp.int32), (coors[1:] != coors[:-1]).astype(jnp.int32)])
    unq_inv = (jnp.cumsum(is_new) - 1).astype(jnp.int32)
    n_act = unq_inv[-1] + 1
    unq = jnp.full((_NV,), coors[0], coors.dtype).at[unq_inv].set(coors)
    seg_starts = jnp.searchsorted(coors, unq).astype(jnp.int32)
    seg_t = unq_inv.reshape(1, n)

    pts_t = points.T
    fcl_t = f_cluster.T
    f32 = jnp.float32
    out32 = jax.ShapeDtypeStruct((32, n), f32)

    def block(p, feats_a, feats_b):
        has_b = feats_b is not None
        wa = feats_a.shape[0]
        w0 = p['W0']
        w0p, w0a = w0[:4].T, w0[4:4 + wa].T
        w0b = w0[4 + wa:].T if has_b else jnp.zeros((32, 32), f32)
        fb = feats_b if has_b else jnp.zeros((32, n), f32)
        col = lambda v: v.reshape(-1, 1)

        x, fx = pl.pallas_call(
            lambda *rs: _stage1_fwd_kernel(has_b, *rs),
            grid=(nc,),
            in_specs=[_chunk(1, False, nc), _chunk(4, False, nc),
                      _chunk(wa, False, nc), _chunk(32, False, nc),
                      _chunk(3, False, nc),
                      _full((32, 4)), _full((32, wa)), _full((32, 32)),
                      _full((32, 1)), _full((32, 1)), _full((32, 1)),
                      _full((16, 3)), _full((16, 1)), _full((32, 16)),
                      _full((32, 1))],
            out_specs=[_chunk(32, False, nc), _chunk(32, False, nc)],
            out_shape=[out32, out32],
            scratch_shapes=_scratch(),
            compiler_params=_cparams(),
        )(seg_t, pts_t, feats_a, fb, fcl_t,
          w0p, w0a, w0b, col(p['b0']), col(p['g0']), col(p['bt0']),
          p['rel_W1'].T, col(p['rel_b1']), p['rel_W2'].T, col(p['rel_b2']))

        y = pl.pallas_call(
            _stage2_bwd_kernel,
            grid=(nc,),
            in_specs=[_chunk(1, True, nc), _chunk(32, True, nc),
                      _chunk(32, True, nc),
                      _full((32, 32)), _full((32, 32)),
                      _full((32, 1)), _full((32, 1)), _full((32, 1))],
            out_specs=[_chunk(32, True, nc)],
            out_shape=[out32],
            scratch_shapes=_scratch(),
            compiler_params=_cparams(),
        )(seg_t, x, fx, p['W1'][:32].T, p['W1'][32:].T,
          col(p['b1']), col(p['g1']), col(p['bt1']))[0]

        fy = pl.pallas_call(
            _fwd_scan_kernel,
            grid=(nc,),
            in_specs=[_chunk(1, False, nc), _chunk(32, False, nc)],
            out_specs=[_chunk(32, False, nc)],
            out_shape=[out32],
            scratch_shapes=_scratch(),
            compiler_params=_cparams(),
        )(seg_t, y)[0]

        c1 = pl.pallas_call(
            _bwd_combine_kernel,
            grid=(nc,),
            in_specs=[_chunk(1, True, nc), _chunk(32, True, nc),
                      _chunk(32, True, nc)],
            out_specs=[_chunk(32, True, nc)],
            out_shape=[out32],
            scratch_shapes=_scratch(),
            compiler_params=_cparams(),
        )(seg_t, y, fy)[0]

        return y, c1

    feats_a, feats_b = features.T, None
    pooled_pts = []
    for p in params:
        y, c1 = block(p, feats_a, feats_b)
        pooled_pts.append(c1)
        feats_a, feats_b = y, c1

    out_feats = jnp.concatenate([feats_a, feats_b], axis=0).T
    pooled = jnp.concatenate(
        [jnp.take(c, seg_starts, axis=1) for c in pooled_pts], axis=0).T
    mask = (jnp.arange(_NV) < n_act)[:, None]
    final_cluster_feats = jnp.where(mask, pooled, -jnp.inf)
    return (out_feats, final_cluster_feats, unq)


# speedup vs baseline: 2.8720x; 1.0582x over previous
import jax
import jax.numpy as jnp
from jax.experimental import pallas as pl
from jax.experimental.pallas import tpu as pltpu

_EPS = 1e-3
_P = 6400
_NV = 10000
_NEG = float(jnp.finfo(jnp.float32).min)


def _ln_t(x, g, b):
    m = x.mean(0, keepdims=True)
    v = ((x - m) ** 2).mean(0, keepdims=True)
    return (x - m) / jnp.sqrt(v + _EPS) * g + b


def _seg_scan_max_t(x, seg, reverse):
    f, n = x.shape
    d = 1
    while d < n:
        if not reverse:
            xs = jnp.concatenate([jnp.full((f, d), _NEG, x.dtype), x[:, :-d]], 1)
            ss = jnp.concatenate([jnp.full((1, d), -1, seg.dtype), seg[:, :-d]], 1)
        else:
            xs = jnp.concatenate([x[:, d:], jnp.full((f, d), _NEG, x.dtype)], 1)
            ss = jnp.concatenate([seg[:, d:], jnp.full((1, d), -1, seg.dtype)], 1)
        x = jnp.maximum(x, jnp.where(ss == seg, xs, _NEG))
        d *= 2
    return x


def _init_carry(cvec, cseg):
    @pl.when(pl.program_id(0) == 0)
    def _():
        cvec[...] = jnp.full(cvec.shape, _NEG, jnp.float32)
        cseg[0] = -1
        cseg[1] = -1


def _stage1(pts, a, b, fcl, w0p, w0a, w0b, b0, g0, bt0, rw1, rb1, rw2, rb2):
    dot = lambda w, v: jnp.dot(w, v, preferred_element_type=jnp.float32)
    pre = dot(w0p, pts) + dot(w0a, a)
    if b is not None:
        pre = pre + dot(w0b, b)
    pre = pre + b0
    x = jnp.maximum(_ln_t(pre, g0, bt0), 0.0)
    rel = jnp.maximum(dot(rw1, fcl / 10.0) + rb1, 0.0)
    rel = dot(rw2, rel) + rb2
    return x + rel


def _entry_fwd_kernel(seg_ref, pts_ref, a_ref, fcl_ref,
                      w0p_ref, w0a_ref, b0_ref, g0_ref, bt0_ref,
                      rw1_ref, rb1_ref, rw2_ref, rb2_ref,
                      x_out, fx_out, cvec, cseg):
    _init_carry(cvec, cseg)
    seg = seg_ref[...]
    x = _stage1(pts_ref[...], a_ref[...], None, fcl_ref[...],
                w0p_ref[...], w0a_ref[...], None,
                b0_ref[...], g0_ref[...], bt0_ref[...],
                rw1_ref[...], rb1_ref[...], rw2_ref[...], rb2_ref[...])
    x_out[...] = x
    fx = _seg_scan_max_t(x, seg, reverse=False)
    fx = jnp.where(seg == cseg[0], jnp.maximum(fx, cvec[:, :1]), fx)
    fx_out[...] = fx
    cvec[:, :1] = fx[:, -1:]
    cseg[0] = seg[0, -1]


def _rev_kernel(seg_ref, x_ref, fx_ref,
                w1a_ref, w1b_ref, b1_ref, g1_ref, bt1_ref,
                y_out, by_out, cvec, cseg):
    _init_carry(cvec, cseg)
    seg = seg_ref[...]
    x = x_ref[...]
    bx = _seg_scan_max_t(x, seg, reverse=True)
    bx = jnp.where(seg == cseg[0], jnp.maximum(bx, cvec[:, :1]), bx)
    cvec[:, :1] = bx[:, :1]
    cseg[0] = seg[0, 0]

    c0 = jnp.maximum(fx_ref[...], bx)
    dot = lambda w, v: jnp.dot(w, v, preferred_element_type=jnp.float32)
    pre = dot(w1a_ref[...], x) + dot(w1b_ref[...], c0) + b1_ref[...]
    y = jnp.maximum(_ln_t(pre, g1_ref[...], bt1_ref[...]), 0.0)
    y_out[...] = y

    by = _seg_scan_max_t(y, seg, reverse=True)
    by = jnp.where(seg == cseg[1], jnp.maximum(by, cvec[:, 1:]), by)
    by_out[...] = by
    cvec[:, 1:] = by[:, :1]
    cseg[1] = seg[0, 0]


def _fwd_next_kernel(seg_ref, y_ref, by_ref, pts_ref, fcl_ref,
                     w0p_ref, w0y_ref, w0c_ref, b0_ref, g0_ref, bt0_ref,
                     rw1_ref, rb1_ref, rw2_ref, rb2_ref,
                     c_out, x_out, fx_out, cvec, cseg):
    _init_carry(cvec, cseg)
    seg = seg_ref[...]
    y = y_ref[...]
    fy = _seg_scan_max_t(y, seg, reverse=False)
    fy = jnp.where(seg == cseg[0], jnp.maximum(fy, cvec[:, :1]), fy)
    cvec[:, :1] = fy[:, -1:]
    cseg[0] = seg[0, -1]
    c1 = jnp.maximum(fy, by_ref[...])
    c_out[...] = c1

    x = _stage1(pts_ref[...], y, c1, fcl_ref[...],
                w0p_ref[...], w0y_ref[...], w0c_ref[...],
                b0_ref[...], g0_ref[...], bt0_ref[...],
                rw1_ref[...], rb1_ref[...], rw2_ref[...], rb2_ref[...])
    x_out[...] = x
    fx = _seg_scan_max_t(x, seg, reverse=False)
    fx = jnp.where(seg == cseg[1], jnp.maximum(fx, cvec[:, 1:]), fx)
    fx_out[...] = fx
    cvec[:, 1:] = fx[:, -1:]
    cseg[1] = seg[0, -1]


def _fwd_final_kernel(seg_ref, y_ref, by_ref, c_out, cvec, cseg):
    _init_carry(cvec, cseg)
    seg = seg_ref[...]
    fy = _seg_scan_max_t(y_ref[...], seg, reverse=False)
    fy = jnp.where(seg == cseg[0], jnp.maximum(fy, cvec[:, :1]), fy)
    cvec[:, :1] = fy[:, -1:]
    cseg[0] = seg[0, -1]
    c_out[...] = jnp.maximum(fy, by_ref[...])


def _full(shape):
    return pl.BlockSpec(shape, lambda c: tuple(0 for _ in shape))


def _chunk(height, rev, nc):
    if rev:
        return pl.BlockSpec((height, _P), lambda c: (0, nc - 1 - c))
    return pl.BlockSpec((height, _P), lambda c: (0, c))


def _scratch():
    return [pltpu.VMEM((32, 2), jnp.float32), pltpu.SMEM((2,), jnp.int32)]


def _cparams():
    return pltpu.CompilerParams(dimension_semantics=("arbitrary",))


def kernel(points, features, coors, f_cluster, params):
    n = points.shape[0]
    nc = n // _P
    coors = coors.astype(jnp.int32)

    is_new = jnp.concatenate(
        [jnp.ones((1,), jnp.int32), (coors[1:] != coors[:-1]).astype(jnp.int32)])
    unq_inv = (jnp.cumsum(is_new) - 1).astype(jnp.int32)
    n_act = unq_inv[-1] + 1
    unq = jnp.full((_NV,), coors[0], coors.dtype).at[unq_inv].set(coors)
    seg_starts = jnp.searchsorted(coors, unq).astype(jnp.int32)
    seg_t = unq_inv.reshape(1, n)

    pts_t = points.T
    fcl_t = f_cluster.T
    f32 = jnp.float32
    out32 = jax.ShapeDtypeStruct((32, n), f32)
    col = lambda v: v.reshape(-1, 1)

    def rel_args(p):
        return (p['rel_W1'].T, col(p['rel_b1']), p['rel_W2'].T, col(p['rel_b2']))

    def ln0_args(p):
        return (col(p['b0']), col(p['g0']), col(p['bt0']))

    p0 = params[0]
    x, fx = pl.pallas_call(
        _entry_fwd_kernel,
        grid=(nc,),
        in_specs=[_chunk(1, False, nc), _chunk(4, False, nc),
                  _chunk(12, False, nc), _chunk(3, False, nc),
                  _full((32, 4)), _full((32, 12)),
                  _full((32, 1)), _full((32, 1)), _full((32, 1)),
                  _full((16, 3)), _full((16, 1)), _full((32, 16)),
                  _full((32, 1))],
        out_specs=[_chunk(32, False, nc), _chunk(32, False, nc)],
        out_shape=[out32, out32],
        scratch_shapes=_scratch(),
        compiler_params=_cparams(),
    )(seg_t, pts_t, features.T, fcl_t,
      p0['W0'][:4].T, p0['W0'][4:16].T, *ln0_args(p0), *rel_args(p0))

    def rev(p, x, fx):
        return pl.pallas_call(
            _rev_kernel,
            grid=(nc,),
            in_specs=[_chunk(1, True, nc), _chunk(32, True, nc),
                      _chunk(32, True, nc),
                      _full((32, 32)), _full((32, 32)),
                      _full((32, 1)), _full((32, 1)), _full((32, 1))],
            out_specs=[_chunk(32, True, nc), _chunk(32, True, nc)],
            out_shape=[out32, out32],
            scratch_shapes=_scratch(),
            compiler_params=_cparams(),
        )(seg_t, x, fx, p['W1'][:32].T, p['W1'][32:].T,
          col(p['b1']), col(p['g1']), col(p['bt1']))

    def fwd_next(pn, y, by):
        return pl.pallas_call(
            _fwd_next_kernel,
            grid=(nc,),
            in_specs=[_chunk(1, False, nc), _chunk(32, False, nc),
                      _chunk(32, False, nc), _chunk(4, False, nc),
                      _chunk(3, False, nc),
                      _full((32, 4)), _full((32, 32)), _full((32, 32)),
                      _full((32, 1)), _full((32, 1)), _full((32, 1)),
                      _full((16, 3)), _full((16, 1)), _full((32, 16)),
                      _full((32, 1))],
            out_specs=[_chunk(32, False, nc)] * 3,
            out_shape=[out32, out32, out32],
            scratch_shapes=_scratch(),
            compiler_params=_cparams(),
        )(seg_t, y, by, pts_t, fcl_t,
          pn['W0'][:4].T, pn['W0'][4:36].T, pn['W0'][36:].T,
          *ln0_args(pn), *rel_args(pn))

    pooled_pts = []
    y, by = rev(params[0], x, fx)
    c1, x, fx = fwd_next(params[1], y, by)
    pooled_pts.append(c1)
    y, by = rev(params[1], x, fx)
    c1, x, fx = fwd_next(params[2], y, by)
    pooled_pts.append(c1)
    y, by = rev(params[2], x, fx)
    c1 = pl.pallas_call(
        _fwd_final_kernel,
        grid=(nc,),
        in_specs=[_chunk(1, False, nc), _chunk(32, False, nc),
                  _chunk(32, False, nc)],
        out_specs=[_chunk(32, False, nc)],
        out_shape=[out32],
        scratch_shapes=_scratch(),
        compiler_params=_cparams(),
    )(seg_t, y, by)[0]
    pooled_pts.append(c1)

    out_feats = jnp.concatenate([y, c1], axis=0).T
    pooled = jnp.concatenate(
        [jnp.take(c, seg_starts, axis=1) for c in pooled_pts], axis=0).T
    mask = (jnp.arange(_NV) < n_act)[:, None]
    final_cluster_feats = jnp.where(mask, pooled, -jnp.inf)
    return (out_feats, final_cluster_feats, unq)
